# baseline (speedup 1.0000x reference)
import functools

import jax
import jax.numpy as jnp
from jax import lax
from jax.experimental import pallas as pl
from jax.experimental.pallas import tpu as pltpu
from jax.experimental.pallas import tpu_sc as plsc

BATCH = 16384
HIST = 200
EMBED = 64

BB = 256
NG = 2


def _build():
    info = plsc.get_sparse_core_info()
    nc, ns = info.num_cores, info.num_subcores
    nw = nc * ns
    bpw = BATCH // nw
    steps = HIST * (bpw // BB)

    mesh = plsc.VectorSubcoreMesh(core_axis_name="c", subcore_axis_name="s")

    @functools.partial(
        pl.kernel,
        mesh=mesh,
        out_type=jax.ShapeDtypeStruct((HIST, 8, 128, 8, 128), jnp.float32),
        scratch_types=[
            pltpu.VMEM((BB,), jnp.int32),
            pltpu.VMEM((BB,), jnp.int32),
            pltpu.VMEM((BB, EMBED), jnp.float32),
            pltpu.VMEM((BB, EMBED), jnp.float32),
            pltpu.VMEM((8, 2, 8, 128), jnp.float32),
            pltpu.VMEM((8, 2, 8, 128), jnp.float32),
            pltpu.SemaphoreType.DMA,
            pltpu.SemaphoreType.DMA,
            pltpu.SemaphoreType.DMA,
            pltpu.SemaphoreType.DMA,
            pltpu.SemaphoreType.DMA,
            pltpu.SemaphoreType.DMA,
        ],
        compiler_params=pltpu.CompilerParams(
            use_tc_tiling_on_sc=False, needs_layout_passes=False),
    )
    def gather_kernel(xt_hbm, table_hbm, out_hbm,
                      idx0, idx1, sb0, sb1, tb0, tb1,
                      asem0, asem1, gsem0, gsem1, wsem0, wsem1):
        wid = lax.axis_index("s") * nc + lax.axis_index("c")
        b_base = wid * bpw
        bo_base = wid * (bpw // 128)

        idx_b = (idx0, idx1)
        sb_b = (sb0, sb1)
        tb_b = (tb0, tb1)
        asem = (asem0, asem1)
        gsem = (gsem0, gsem1)
        wsem = (wsem0, wsem1)

        iota = lax.iota(jnp.int32, 16)

        def fire_idx(t, b):
            h = t // 2
            j = t % 2
            pltpu.async_copy(
                xt_hbm.at[h, pl.ds(b_base + j * BB, BB)], idx_b[b], asem[b])

        def wait_idx(b):
            pltpu.make_async_copy(
                xt_hbm.at[0, pl.ds(0, BB)], idx_b[b], asem[b]).wait()

        def fire_gather(b):
            for k in range(NG):
                pltpu.async_copy(
                    table_hbm.at[idx_b[b].at[pl.ds(k * 128, 128)]],
                    sb_b[b].at[pl.ds(k * 128, 128)],
                    gsem[b],
                )

        def wait_gather(b):
            pltpu.make_async_copy(
                table_hbm.at[pl.ds(0, BB)], sb_b[b], gsem[b]).wait()

        def transpose(b):
            sb = sb_b[b]
            tb = tb_b[b]

            def q_body(q, carry):
                eo = q // 2
                b2 = q % 2
                rows0 = b2 * 128 + iota
                for ei in range(8):
                    col = jnp.full((16,), 0, jnp.int32) + (eo * 8 + ei)
                    for bg in range(8):
                        v = plsc.load_gather(sb, [rows0 + bg * 16, col])
                        tb[eo, b2, ei, pl.ds(bg * 16, 16)] = v
                return carry

            lax.fori_loop(0, 16, q_body, 0)

        def fire_write(t, b):
            h = t // 2
            j = t % 2
            pltpu.async_copy(
                tb_b[b], out_hbm.at[h, :, pl.ds(bo_base + j * 2, 2)],
                wsem[b])

        def wait_write(b):
            pltpu.make_async_copy(
                tb_b[b], out_hbm.at[0, :, pl.ds(0, 2)], wsem[b]).wait()

        def slot(t, b, first=False, last=False):
            if not first:
                wait_write(b)
            wait_idx(b)
            fire_gather(b)
            if not first:
                wait_gather(1 - b)
                transpose(1 - b)
                fire_write(t - 1, 1 - b)
            if not last:
                fire_idx(t + 1, 1 - b)

        fire_idx(0, 0)
        wait_idx(0)
        fire_gather(0)
        fire_idx(1, 1)
        wait_idx(1)
        fire_gather(1)
        wait_gather(0)
        transpose(0)
        fire_write(0, 0)
        fire_idx(2, 0)

        def pair(s, carry):
            t = 2 * s
            slot(t, 0)
            slot(t + 1, 1)
            return carry

        lax.fori_loop(1, steps // 2 - 1, pair, 0)

        slot(steps - 2, 0)
        slot(steps - 1, 1, last=True)

        wait_gather(1)
        transpose(1)
        fire_write(steps - 1, 1)
        wait_write(0)
        wait_write(1)

    return gather_kernel


_GATHER = _build()


@jax.jit
def kernel(x, table):
    xt = x.T.astype(jnp.int32)
    out5 = _GATHER(xt, table)
    return out5.transpose(2, 4, 0, 1, 3).reshape(BATCH, HIST, EMBED)


# speedup vs baseline: 2.5888x; 2.5888x over previous
import functools

import jax
import jax.numpy as jnp
from jax import lax
from jax.experimental import pallas as pl
from jax.experimental.pallas import tpu as pltpu
from jax.experimental.pallas import tpu_sc as plsc

BATCH = 16384
HIST = 200
EMBED = 64

BB = 256
NG = 2
TP = 130


def _build():
    info = plsc.get_sparse_core_info()
    nc, ns = info.num_cores, info.num_subcores
    nw = nc * ns
    bpw = BATCH // nw
    steps = HIST * (bpw // BB)

    mesh = plsc.VectorSubcoreMesh(core_axis_name="c", subcore_axis_name="s")

    @functools.partial(
        pl.kernel,
        mesh=mesh,
        out_type=jax.ShapeDtypeStruct((HIST, 8, 128, 8, 128), jnp.float32),
        scratch_types=[
            pltpu.VMEM((BB,), jnp.int32),
            pltpu.VMEM((BB,), jnp.int32),
            pltpu.VMEM((BB, EMBED), jnp.float32),
            pltpu.VMEM((BB, EMBED), jnp.float32),
            pltpu.VMEM((8, 2, 8, TP), jnp.float32),
            pltpu.VMEM((8, 2, 8, TP), jnp.float32),
            pltpu.SemaphoreType.DMA,
            pltpu.SemaphoreType.DMA,
            pltpu.SemaphoreType.DMA,
            pltpu.SemaphoreType.DMA,
            pltpu.SemaphoreType.DMA,
            pltpu.SemaphoreType.DMA,
        ],
        compiler_params=pltpu.CompilerParams(
            use_tc_tiling_on_sc=False, needs_layout_passes=False),
    )
    def gather_kernel(xt_hbm, table_hbm, out_hbm,
                      idx0, idx1, sb0, sb1, tb0, tb1,
                      asem0, asem1, gsem0, gsem1, wsem0, wsem1):
        wid = lax.axis_index("s") * nc + lax.axis_index("c")
        b_base = wid * bpw
        bo_base = wid * (bpw // 128)

        idx_b = (idx0, idx1)
        sb_b = (sb0, sb1)
        tb_b = (tb0, tb1)
        asem = (asem0, asem1)
        gsem = (gsem0, gsem1)
        wsem = (wsem0, wsem1)

        iota = lax.iota(jnp.int32, 16)
        i_hi = lax.shift_right_logical(iota, 3)
        i_lo = lax.bitwise_and(iota, 7)
        zeros16 = jnp.full((16,), 0, jnp.int32)

        def fire_idx(t, b):
            h = t // 2
            j = t % 2
            pltpu.async_copy(
                xt_hbm.at[h, pl.ds(b_base + j * BB, BB)], idx_b[b], asem[b])

        def wait_idx(b):
            pltpu.make_async_copy(
                xt_hbm.at[0, pl.ds(0, BB)], idx_b[b], asem[b]).wait()

        def fire_gather(b):
            for k in range(NG):
                pltpu.async_copy(
                    table_hbm.at[idx_b[b].at[pl.ds(k * 128, 128)]],
                    sb_b[b].at[pl.ds(k * 128, 128)],
                    gsem[b],
                )

        def wait_gather(b):
            pltpu.make_async_copy(
                table_hbm.at[pl.ds(0, BB)], sb_b[b], gsem[b]).wait()

        def transpose(b):
            sb = sb_b[b]
            tb = tb_b[b]

            def r_body(r, carry):
                b2v = zeros16 + lax.shift_right_logical(r, 7)
                biv = zeros16 + lax.bitwise_and(r, 127)
                for g in range(4):
                    v = sb[r, pl.ds(g * 16, 16)]
                    plsc.store_scatter(
                        tb, [i_hi + 2 * g, b2v, i_lo, biv], v)
                return carry

            lax.fori_loop(0, BB, r_body, 0)

        def fire_write(t, b):
            h = t // 2
            j = t % 2
            pltpu.async_copy(
                tb_b[b].at[:, :, :, pl.ds(0, 128)],
                out_hbm.at[h, :, pl.ds(bo_base + j * 2, 2)],
                wsem[b])

        def wait_write(b):
            pltpu.make_async_copy(
                tb_b[b].at[:, :, :, pl.ds(0, 128)],
                out_hbm.at[0, :, pl.ds(0, 2)], wsem[b]).wait()

        def slot(t, b, first=False, last=False):
            if not first:
                wait_write(b)
            wait_idx(b)
            fire_gather(b)
            if not first:
                wait_gather(1 - b)
                transpose(1 - b)
                fire_write(t - 1, 1 - b)
            if not last:
                fire_idx(t + 1, 1 - b)

        fire_idx(0, 0)
        wait_idx(0)
        fire_gather(0)
        fire_idx(1, 1)
        wait_idx(1)
        fire_gather(1)
        wait_gather(0)
        transpose(0)
        fire_write(0, 0)
        fire_idx(2, 0)

        def pair(s, carry):
            t = 2 * s
            slot(t, 0)
            slot(t + 1, 1)
            return carry

        lax.fori_loop(1, steps // 2 - 1, pair, 0)

        slot(steps - 2, 0)
        slot(steps - 1, 1, last=True)

        wait_gather(1)
        transpose(1)
        fire_write(steps - 1, 1)
        wait_write(0)
        wait_write(1)

    return gather_kernel


_GATHER = _build()


@jax.jit
def kernel(x, table):
    xt = x.T.astype(jnp.int32)
    out5 = _GATHER(xt, table)
    return out5.transpose(2, 4, 0, 1, 3).reshape(BATCH, HIST, EMBED)


# speedup vs baseline: 2.6655x; 1.0296x over previous
import functools

import jax
import jax.numpy as jnp
from jax import lax
from jax.experimental import pallas as pl
from jax.experimental.pallas import tpu as pltpu
from jax.experimental.pallas import tpu_sc as plsc

BATCH = 16384
HIST = 200
EMBED = 64

BB = 256
NG = 2
TP = 130


def _build():
    info = plsc.get_sparse_core_info()
    nc, ns = info.num_cores, info.num_subcores
    nw = nc * ns
    bpw = BATCH // nw
    steps = HIST * (bpw // BB)

    mesh = plsc.VectorSubcoreMesh(core_axis_name="c", subcore_axis_name="s")

    @functools.partial(
        pl.kernel,
        mesh=mesh,
        out_type=jax.ShapeDtypeStruct((HIST, 8, 128, 8, 128), jnp.float32),
        scratch_types=[
            pltpu.VMEM((BB,), jnp.int32),
            pltpu.VMEM((BB,), jnp.int32),
            pltpu.VMEM((BB, EMBED), jnp.float32),
            pltpu.VMEM((BB, EMBED), jnp.float32),
            pltpu.VMEM((8, 2, 8, TP), jnp.float32),
            pltpu.VMEM((8, 2, 8, TP), jnp.float32),
            pltpu.SemaphoreType.DMA,
            pltpu.SemaphoreType.DMA,
            pltpu.SemaphoreType.DMA,
            pltpu.SemaphoreType.DMA,
            pltpu.SemaphoreType.DMA,
            pltpu.SemaphoreType.DMA,
        ],
        compiler_params=pltpu.CompilerParams(
            use_tc_tiling_on_sc=False, needs_layout_passes=False),
    )
    def gather_kernel(xt_hbm, table_hbm, out_hbm,
                      idx0, idx1, sb0, sb1, tb0, tb1,
                      asem0, asem1, gsem0, gsem1, wsem0, wsem1):
        wid = lax.axis_index("s") * nc + lax.axis_index("c")
        b_base = wid * bpw
        bo_base = wid * (bpw // 128)

        idx_b = (idx0, idx1)
        sb_b = (sb0, sb1)
        tb_b = (tb0, tb1)
        asem = (asem0, asem1)
        gsem = (gsem0, gsem1)
        wsem = (wsem0, wsem1)

        iota = lax.iota(jnp.int32, 16)
        i_hi = lax.shift_right_logical(iota, 3)
        i_lo = lax.bitwise_and(iota, 7)
        eo_g = tuple(i_hi + 2 * g for g in range(4))
        zeros16 = jnp.full((16,), 0, jnp.int32)

        def fire_idx(t, b):
            h = t // 2
            j = t % 2
            pltpu.async_copy(
                xt_hbm.at[h, pl.ds(b_base + j * BB, BB)], idx_b[b], asem[b])

        def wait_idx(b):
            pltpu.make_async_copy(
                xt_hbm.at[0, pl.ds(0, BB)], idx_b[b], asem[b]).wait()

        def fire_gather(b):
            for k in range(NG):
                pltpu.async_copy(
                    table_hbm.at[idx_b[b].at[pl.ds(k * 128, 128)]],
                    sb_b[b].at[pl.ds(k * 128, 128)],
                    gsem[b],
                )

        def wait_gather(b):
            pltpu.make_async_copy(
                table_hbm.at[pl.ds(0, BB)], sb_b[b], gsem[b]).wait()

        def transpose(b):
            sb = sb_b[b]
            tb = tb_b[b]

            def r_body(p, carry):
                r0 = p * 2
                for u in range(2):
                    r = r0 + u
                    b2v = zeros16 + lax.shift_right_logical(r, 7)
                    biv = zeros16 + lax.bitwise_and(r, 127)
                    for g in range(4):
                        v = sb[r, pl.ds(g * 16, 16)]
                        plsc.store_scatter(
                            tb, [eo_g[g], b2v, i_lo, biv], v)
                return carry

            lax.fori_loop(0, BB // 2, r_body, 0)

        def fire_write(t, b):
            h = t // 2
            j = t % 2
            pltpu.async_copy(
                tb_b[b].at[:, :, :, pl.ds(0, 128)],
                out_hbm.at[h, :, pl.ds(bo_base + j * 2, 2)],
                wsem[b])

        def wait_write(b):
            pltpu.make_async_copy(
                tb_b[b].at[:, :, :, pl.ds(0, 128)],
                out_hbm.at[0, :, pl.ds(0, 2)], wsem[b]).wait()

        def slot(t, b, first=False, last=False):
            if not first:
                wait_write(b)
            wait_idx(b)
            fire_gather(b)
            if not first:
                wait_gather(1 - b)
                transpose(1 - b)
                fire_write(t - 1, 1 - b)
            if not last:
                fire_idx(t + 1, 1 - b)

        fire_idx(0, 0)
        wait_idx(0)
        fire_gather(0)
        fire_idx(1, 1)
        wait_idx(1)
        fire_gather(1)
        wait_gather(0)
        transpose(0)
        fire_write(0, 0)
        fire_idx(2, 0)

        def pair(s, carry):
            t = 2 * s
            slot(t, 0)
            slot(t + 1, 1)
            return carry

        lax.fori_loop(1, steps // 2 - 1, pair, 0)

        slot(steps - 2, 0)
        slot(steps - 1, 1, last=True)

        wait_gather(1)
        transpose(1)
        fire_write(steps - 1, 1)
        wait_write(0)
        wait_write(1)

    return gather_kernel


_GATHER = _build()


@jax.jit
def kernel(x, table):
    xt = x.T.astype(jnp.int32)
    out5 = _GATHER(xt, table)
    return out5.transpose(2, 4, 0, 1, 3).reshape(BATCH, HIST, EMBED)


# speedup vs baseline: 2.7059x; 1.0151x over previous
import functools

import jax
import jax.numpy as jnp
from jax import lax
from jax.experimental import pallas as pl
from jax.experimental.pallas import tpu as pltpu
from jax.experimental.pallas import tpu_sc as plsc

BATCH = 16384
HIST = 200
EMBED = 64

BB = 256
NG = 2
TP = 130


def _build():
    info = plsc.get_sparse_core_info()
    nc, ns = info.num_cores, info.num_subcores
    nw = nc * ns
    bpw = BATCH // nw
    steps = HIST * (bpw // BB)

    mesh = plsc.VectorSubcoreMesh(core_axis_name="c", subcore_axis_name="s")

    @functools.partial(
        pl.kernel,
        mesh=mesh,
        out_type=jax.ShapeDtypeStruct((HIST, 8, 128, 8, 128), jnp.float32),
        scratch_types=[
            pltpu.VMEM((BB,), jnp.int32),
            pltpu.VMEM((BB,), jnp.int32),
            pltpu.VMEM((BB, EMBED), jnp.float32),
            pltpu.VMEM((BB, EMBED), jnp.float32),
            pltpu.VMEM((8, 2, 8, TP), jnp.float32),
            pltpu.VMEM((8, 2, 8, TP), jnp.float32),
            pltpu.SemaphoreType.DMA,
            pltpu.SemaphoreType.DMA,
            pltpu.SemaphoreType.DMA,
            pltpu.SemaphoreType.DMA,
            pltpu.SemaphoreType.DMA,
            pltpu.SemaphoreType.DMA,
        ],
        compiler_params=pltpu.CompilerParams(
            use_tc_tiling_on_sc=False, needs_layout_passes=False),
    )
    def gather_kernel(xt_hbm, table_hbm, out_hbm,
                      idx0, idx1, sb0, sb1, tb0, tb1,
                      asem0, asem1, gsem0, gsem1, wsem0, wsem1):
        wid = lax.axis_index("s") * nc + lax.axis_index("c")
        b_base = wid * bpw
        bo_base = wid * (bpw // 128)

        idx_b = (idx0, idx1)
        sb_b = (sb0, sb1)
        tb_b = (tb0, tb1)
        asem = (asem0, asem1)
        gsem = (gsem0, gsem1)
        wsem = (wsem0, wsem1)

        iota = lax.iota(jnp.int32, 16)
        i_hi = lax.shift_right_logical(iota, 3)
        i_lo = lax.bitwise_and(iota, 7)
        eo_g = tuple(i_hi + 2 * g for g in range(4))
        zeros16 = jnp.full((16,), 0, jnp.int32)

        def fire_idx(t, b):
            h = t // 2
            j = t % 2
            pltpu.async_copy(
                xt_hbm.at[h, pl.ds(b_base + j * BB, BB)], idx_b[b], asem[b])

        def wait_idx(b):
            pltpu.make_async_copy(
                xt_hbm.at[0, pl.ds(0, BB)], idx_b[b], asem[b]).wait()

        def fire_gather(b):
            for k in range(NG):
                pltpu.async_copy(
                    table_hbm.at[idx_b[b].at[pl.ds(k * 128, 128)]],
                    sb_b[b].at[pl.ds(k * 128, 128)],
                    gsem[b],
                )

        def wait_gather(b):
            pltpu.make_async_copy(
                table_hbm.at[pl.ds(0, BB)], sb_b[b], gsem[b]).wait()

        def transpose(b):
            sb = sb_b[b]
            tb = tb_b[b]

            def r_body(p, carry):
                r0 = p * 4
                for u in range(4):
                    r = r0 + u
                    b2v = zeros16 + lax.shift_right_logical(r, 7)
                    biv = zeros16 + lax.bitwise_and(r, 127)
                    for g in range(4):
                        v = sb[r, pl.ds(g * 16, 16)]
                        plsc.store_scatter(
                            tb, [eo_g[g], b2v, i_lo, biv], v)
                return carry

            lax.fori_loop(0, BB // 4, r_body, 0)

        def fire_write(t, b):
            h = t // 2
            j = t % 2
            pltpu.async_copy(
                tb_b[b].at[:, :, :, pl.ds(0, 128)],
                out_hbm.at[h, :, pl.ds(bo_base + j * 2, 2)],
                wsem[b])

        def wait_write(b):
            pltpu.make_async_copy(
                tb_b[b].at[:, :, :, pl.ds(0, 128)],
                out_hbm.at[0, :, pl.ds(0, 2)], wsem[b]).wait()

        def slot(t, b, first=False, last=False):
            if not first:
                wait_write(b)
            wait_idx(b)
            fire_gather(b)
            if not first:
                wait_gather(1 - b)
                transpose(1 - b)
                fire_write(t - 1, 1 - b)
            if not last:
                fire_idx(t + 1, 1 - b)

        fire_idx(0, 0)
        wait_idx(0)
        fire_gather(0)
        fire_idx(1, 1)
        wait_idx(1)
        fire_gather(1)
        wait_gather(0)
        transpose(0)
        fire_write(0, 0)
        fire_idx(2, 0)

        def pair(s, carry):
            t = 2 * s
            slot(t, 0)
            slot(t + 1, 1)
            return carry

        lax.fori_loop(1, steps // 2 - 1, pair, 0)

        slot(steps - 2, 0)
        slot(steps - 1, 1, last=True)

        wait_gather(1)
        transpose(1)
        fire_write(steps - 1, 1)
        wait_write(0)
        wait_write(1)

    return gather_kernel


_GATHER = _build()


@jax.jit
def kernel(x, table):
    xt = x.T.astype(jnp.int32)
    out5 = _GATHER(xt, table)
    return out5.transpose(2, 4, 0, 1, 3).reshape(BATCH, HIST, EMBED)
